# baseline (speedup 1.0000x reference)
import functools

import jax
import jax.numpy as jnp
from jax import lax
from jax.experimental import pallas as pl
from jax.experimental.pallas import tpu as pltpu
from jax.experimental.pallas import tpu_sc as plsc

VOCAB = 1000000
D = 64
BATCH = 4096
HIST = 200

NC = 2
NS = 16
NW = NC * NS

TOTAL = BATCH * HIST
PER_W = TOTAL // NW
GROUP = 512
NG = PER_W // GROUP
NBUF = 3
LOOKAHEAD = 2


def _body(table_hbm, idx_hbm, out_hbm, idx_v, rows_v, *sems):
    gsems = sems[:NBUF]
    osems = sems[NBUF:]
    wid = lax.axis_index("s") * NC + lax.axis_index("c")
    base = wid * PER_W

    pltpu.sync_copy(idx_hbm.at[pl.ds(base, PER_W)], idx_v)

    def start_gather(g, b):
        pltpu.async_copy(
            table_hbm.at[idx_v.at[pl.ds(g * GROUP, GROUP)]],
            rows_v.at[b], gsems[b])

    def wait_gather(g, b):
        pltpu.make_async_copy(
            table_hbm.at[idx_v.at[pl.ds(g * GROUP, GROUP)]],
            rows_v.at[b], gsems[b]).wait()

    def start_write(g, b):
        pltpu.async_copy(
            rows_v.at[b], out_hbm.at[pl.ds(base + g * GROUP, GROUP)],
            osems[b])

    def wait_write(g, b):
        pltpu.make_async_copy(
            rows_v.at[b], out_hbm.at[pl.ds(base + g * GROUP, GROUP)],
            osems[b]).wait()

    for g in range(LOOKAHEAD):
        start_gather(g, g % NBUF)
    for b in range(NBUF):
        g = b
        wait_gather(g, b)
        start_write(g, b)
        g2 = g + LOOKAHEAD
        if g2 >= LOOKAHEAD:
            if g2 >= NBUF:
                wait_write(g2 - NBUF, g2 % NBUF)
            start_gather(g2, g2 % NBUF)

    def ring(r, carry):
        for b in range(NBUF):
            g = r * NBUF + b
            wait_gather(g, b)
            start_write(g, b)
            g2 = g + LOOKAHEAD
            b2 = (b + LOOKAHEAD) % NBUF
            wait_write(g2 - NBUF, b2)
            start_gather(g2, b2)
        return carry

    nring = NG // NBUF
    lax.fori_loop(1, nring - 1, ring, 0)

    for g in range((nring - 1) * NBUF, NG):
        b = g % NBUF
        wait_gather(g, b)
        start_write(g, b)
        g2 = g + LOOKAHEAD
        if g2 < NG:
            wait_write(g2 - NBUF, g2 % NBUF)
            start_gather(g2, g2 % NBUF)

    for g in range(NG - NBUF, NG):
        wait_write(g, g % NBUF)


@jax.jit
def _run(table, idx):
    mesh = plsc.VectorSubcoreMesh(core_axis_name="c", subcore_axis_name="s")
    fn = functools.partial(
        pl.kernel,
        mesh=mesh,
        out_type=jax.ShapeDtypeStruct((TOTAL, D), jnp.float32),
        scratch_types=[
            pltpu.VMEM((PER_W,), jnp.int32),
            pltpu.VMEM((NBUF, GROUP, D), jnp.float32),
        ] + [pltpu.SemaphoreType.DMA] * (2 * NBUF),
        compiler_params=pltpu.CompilerParams(use_tc_tiling_on_sc=False),
    )(_body)
    return fn(table, idx)


def kernel(x, table):
    idx = x.astype(jnp.int32).reshape(TOTAL)
    out = _run(table, idx)
    return out.reshape(BATCH, HIST, D)


# speedup vs baseline: 1.0056x; 1.0056x over previous
import functools

import jax
import jax.numpy as jnp
from jax import lax
from jax.experimental import pallas as pl
from jax.experimental.pallas import tpu as pltpu
from jax.experimental.pallas import tpu_sc as plsc

VOCAB = 1000000
D = 64
BATCH = 4096
HIST = 200

NC = 2
NS = 16
NW = NC * NS

TOTAL = BATCH * HIST
PER_W = TOTAL // NW
GROUP = 256
NG = PER_W // GROUP
NBUF = 3
LOOKAHEAD = 2


def _body(table_hbm, idx_hbm, out_hbm, idx_v, rows_v, *sems):
    gsems = sems[:NBUF]
    osems = sems[NBUF:]
    wid = lax.axis_index("s") * NC + lax.axis_index("c")
    base = wid * PER_W

    pltpu.sync_copy(idx_hbm.at[pl.ds(base, PER_W)], idx_v)

    def start_gather(g, b):
        pltpu.async_copy(
            table_hbm.at[idx_v.at[pl.ds(g * GROUP, GROUP)]],
            rows_v.at[b], gsems[b])

    def wait_gather(g, b):
        pltpu.make_async_copy(
            table_hbm.at[idx_v.at[pl.ds(g * GROUP, GROUP)]],
            rows_v.at[b], gsems[b]).wait()

    def start_write(g, b):
        pltpu.async_copy(
            rows_v.at[b], out_hbm.at[pl.ds(base + g * GROUP, GROUP)],
            osems[b])

    def wait_write(g, b):
        pltpu.make_async_copy(
            rows_v.at[b], out_hbm.at[pl.ds(base + g * GROUP, GROUP)],
            osems[b]).wait()

    for g in range(LOOKAHEAD):
        start_gather(g, g % NBUF)
    for b in range(NBUF):
        g = b
        wait_gather(g, b)
        start_write(g, b)
        g2 = g + LOOKAHEAD
        if g2 >= LOOKAHEAD:
            if g2 >= NBUF:
                wait_write(g2 - NBUF, g2 % NBUF)
            start_gather(g2, g2 % NBUF)

    def ring(r, carry):
        for b in range(NBUF):
            g = r * NBUF + b
            wait_gather(g, b)
            start_write(g, b)
            g2 = g + LOOKAHEAD
            b2 = (b + LOOKAHEAD) % NBUF
            wait_write(g2 - NBUF, b2)
            start_gather(g2, b2)
        return carry

    nring = NG // NBUF
    lax.fori_loop(1, nring - 1, ring, 0)

    for g in range((nring - 1) * NBUF, NG):
        b = g % NBUF
        wait_gather(g, b)
        start_write(g, b)
        g2 = g + LOOKAHEAD
        if g2 < NG:
            wait_write(g2 - NBUF, g2 % NBUF)
            start_gather(g2, g2 % NBUF)

    for g in range(NG - NBUF, NG):
        wait_write(g, g % NBUF)


@jax.jit
def _run(table, idx):
    mesh = plsc.VectorSubcoreMesh(core_axis_name="c", subcore_axis_name="s")
    fn = functools.partial(
        pl.kernel,
        mesh=mesh,
        out_type=jax.ShapeDtypeStruct((TOTAL, D), jnp.float32),
        scratch_types=[
            pltpu.VMEM((PER_W,), jnp.int32),
            pltpu.VMEM((NBUF, GROUP, D), jnp.float32),
        ] + [pltpu.SemaphoreType.DMA] * (2 * NBUF),
        compiler_params=pltpu.CompilerParams(use_tc_tiling_on_sc=False),
    )(_body)
    return fn(table, idx)


def kernel(x, table):
    idx = x.astype(jnp.int32).reshape(TOTAL)
    out = _run(table, idx)
    return out.reshape(BATCH, HIST, D)
